# baseline (speedup 1.0000x reference)
import functools

import jax
import jax.numpy as jnp
from jax import lax
from jax.experimental import pallas as pl
from jax.experimental.pallas import tpu as pltpu
from jax.experimental.pallas import tpu_sc as plsc

_N0, _N1, _KN, _KP = 10000, 2500, 32, 15
_CIN, _C1, _C2, _NCLS = 128, 64, 128, 19
_EXT0, _EXT1 = 0.05, 0.1

_NW = 32


def _sc_gather_layer(feat, ptsx, ptsy, ptsz, idx_flat, chunk):
    B = idx_flat.shape[0]
    D = feat.shape[1]
    V = ptsx.shape[0]
    b_per_w = B // _NW
    nchunk = b_per_w // chunk
    mesh = plsc.VectorSubcoreMesh(core_axis_name="c", subcore_axis_name="s")
    f32 = jnp.float32

    @functools.partial(
        pl.kernel,
        mesh=mesh,
        compiler_params=pltpu.CompilerParams(needs_layout_passes=False),
        out_type=[
            jax.ShapeDtypeStruct((B, D), f32),
            jax.ShapeDtypeStruct((B,), f32),
            jax.ShapeDtypeStruct((B,), f32),
            jax.ShapeDtypeStruct((B,), f32),
        ],
        scratch_types=[
            pltpu.VMEM((b_per_w,), jnp.int32),
            pltpu.VMEM((chunk, D), f32),
            pltpu.VMEM((chunk, D), f32),
            pltpu.VMEM((b_per_w,), f32),
            pltpu.VMEM((b_per_w,), f32),
            pltpu.VMEM((b_per_w,), f32),
            pltpu.VMEM((V,), f32),
            pltpu.VMEM((V,), f32),
            pltpu.VMEM((V,), f32),
            pltpu.SemaphoreType.DMA,
            pltpu.SemaphoreType.DMA,
            pltpu.SemaphoreType.DMA,
            pltpu.SemaphoreType.DMA,
        ],
    )
    def k(feat_h, px_h, py_h, pz_h, idx_h, of_h, ox_h, oy_h, oz_h,
          idx_v, rows_a, rows_b, xv, yv, zv, ptx_v, pty_v, ptz_v,
          ga, gb, wa, wb_s):
        assert nchunk % 2 == 0 and nchunk >= 4
        wid = lax.axis_index("s") * 2 + lax.axis_index("c")
        base = wid * b_per_w
        pltpu.sync_copy(idx_h.at[pl.ds(base, b_per_w)], idx_v)
        pltpu.sync_copy(px_h, ptx_v)
        pltpu.sync_copy(py_h, pty_v)
        pltpu.sync_copy(pz_h, ptz_v)

        def fire(ci, buf, sem):
            pltpu.async_copy(feat_h.at[idx_v.at[pl.ds(ci * chunk, chunk)]], buf, sem)

        def wait_g(buf, sem):
            pltpu.make_async_copy(feat_h.at[pl.ds(0, chunk)], buf, sem).wait()

        def awb(ci, buf, sem):
            pltpu.async_copy(buf, of_h.at[pl.ds(base + ci * chunk, chunk)], sem)

        def wait_wb(buf, sem):
            pltpu.make_async_copy(buf, of_h.at[pl.ds(base, chunk)], sem).wait()

        fire(0, rows_a, ga)
        fire(1, rows_b, gb)

        def pbody(j, _):
            ivec = idx_v[pl.ds(j * 16, 16)]
            xv[pl.ds(j * 16, 16)] = plsc.load_gather(ptx_v, [ivec])
            yv[pl.ds(j * 16, 16)] = plsc.load_gather(pty_v, [ivec])
            zv[pl.ds(j * 16, 16)] = plsc.load_gather(ptz_v, [ivec])
            return ()

        lax.fori_loop(0, b_per_w // 16, pbody, ())

        def body(i, _):
            c = 2 * i
            wait_g(rows_a, ga)
            awb(c, rows_a, wa)
            wait_g(rows_b, gb)
            awb(c + 1, rows_b, wb_s)
            wait_wb(rows_a, wa)

            @pl.when(c + 2 < nchunk)
            def _():
                fire(c + 2, rows_a, ga)

            wait_wb(rows_b, wb_s)

            @pl.when(c + 3 < nchunk)
            def _():
                fire(c + 3, rows_b, gb)

            return ()

        lax.fori_loop(0, nchunk // 2, body, ())

        pltpu.sync_copy(xv, ox_h.at[pl.ds(base, b_per_w)])
        pltpu.sync_copy(yv, oy_h.at[pl.ds(base, b_per_w)])
        pltpu.sync_copy(zv, oz_h.at[pl.ds(base, b_per_w)])

    return k(feat, ptsx, ptsy, ptsz, idx_flat)


def _sc_gather_rows(table, idx):
    B = idx.shape[0]
    D = table.shape[1]
    b_per_w = B // _NW
    chunk = 80
    nchunk = b_per_w // chunk
    mesh = plsc.VectorSubcoreMesh(core_axis_name="c", subcore_axis_name="s")

    @functools.partial(
        pl.kernel,
        mesh=mesh,
        compiler_params=pltpu.CompilerParams(needs_layout_passes=False),
        out_type=jax.ShapeDtypeStruct((B, D), jnp.float32),
        scratch_types=[
            pltpu.VMEM((chunk,), jnp.int32),
            pltpu.VMEM((chunk, D), jnp.float32),
            pltpu.SemaphoreType.DMA,
        ],
    )
    def k(tab_h, idx_h, out_h, idx_v, rows_v, sem):
        wid = lax.axis_index("s") * 2 + lax.axis_index("c")
        base = wid * b_per_w

        def body(i, _):
            off = base + i * chunk
            pltpu.sync_copy(idx_h.at[pl.ds(off, chunk)], idx_v)
            pltpu.async_copy(tab_h.at[idx_v], rows_v, sem).wait()
            pltpu.sync_copy(rows_v, out_h.at[pl.ds(off, chunk)])
            return ()

        lax.fori_loop(0, nchunk, body, ())

    return k(table, idx)


def _leaky(x):
    return jnp.where(x > 0, x, 0.1 * x)


_GP = 8
_GW = _GP * _KN


def _kpconv_body(nx_ref, px_ref, py_ref, pz_ref, dx_ref, dy_ref, dz_ref,
                 kpt_ref, w_ref, o_ref, wf_scr, *, ext, bn, cpad):
    cin = nx_ref.shape[-1]
    cout = w_ref.shape[-1]
    ngrp = bn // _GP
    kx = kpt_ref[0:_KP, :]
    ky = kpt_ref[_KP:2 * _KP, :]
    kz = kpt_ref[2 * _KP:3 * _KP, :]
    pxr = px_ref[...] - dx_ref[...]
    pyr = py_ref[...] - dy_ref[...]
    pzr = pz_ref[...] - dz_ref[...]
    rows = _KP * _GP
    r8 = lax.broadcasted_iota(jnp.int32, (rows, _GW), 0) % _GP
    c32 = lax.broadcasted_iota(jnp.int32, (rows, _GW), 1) // _KN
    maskf = (r8 == c32).astype(jnp.float32)
    for g in range(ngrp):
        ax = pxr[g:g + 1, :] - kx
        ay = pyr[g:g + 1, :] - ky
        az = pzr[g:g + 1, :] - kz
        d2 = ax * ax + ay * ay + az * az
        w15 = jnp.maximum(0.0, 1.0 - jnp.sqrt(d2) * (1.0 / ext))
        wrep = jnp.broadcast_to(w15[:, None, :], (_KP, _GP, _GW)).reshape(rows, _GW)
        m = wrep * maskf
        nxg = nx_ref[g * _GW:(g + 1) * _GW, :]
        wf_scr[g * rows:(g + 1) * rows, :] = jnp.dot(
            m, nxg, preferred_element_type=jnp.float32)
    acc = jnp.zeros((bn, cout), jnp.float32)
    for p in range(_KP):
        parts = [wf_scr[g * rows + p * _GP: g * rows + (p + 1) * _GP, :]
                 for g in range(ngrp)]
        wfp = jnp.concatenate(parts, axis=0)
        acc = acc + jnp.dot(wfp, w_ref[p * cin:(p + 1) * cin, :],
                            preferred_element_type=jnp.float32)
    res = _leaky(acc)
    if cpad > res.shape[1]:
        res = jnp.concatenate(
            [res, jnp.zeros((bn, cpad - res.shape[1]), jnp.float32)], axis=1)
    o_ref[...] = res


def _kpconv(nx_flat, px, py, pz, dxe, dye, dze, kpt, Wflat, ext, bn, cpad=None):
    nd = nx_flat.shape[0] // _KN
    cin = nx_flat.shape[-1]
    cout = Wflat.shape[-1]
    if cpad is None:
        cpad = cout
    grid = (nd // bn,)
    gb = bn // _GP
    pr = (nd // _GP, _GW)
    args = [nx_flat] + [a.reshape(pr) for a in (px, py, pz, dxe, dye, dze)]
    return pl.pallas_call(
        functools.partial(_kpconv_body, ext=ext, bn=bn, cpad=cpad),
        grid=grid,
        in_specs=[pl.BlockSpec((bn * _KN, cin), lambda i: (i, 0))]
        + [pl.BlockSpec((gb, _GW), lambda i: (i, 0))] * 6
        + [
            pl.BlockSpec((3 * _KP, 1), lambda i: (0, 0)),
            pl.BlockSpec((_KP * cin, cout), lambda i: (0, 0)),
        ],
        out_specs=pl.BlockSpec((bn, cpad), lambda i: (i, 0)),
        out_shape=jax.ShapeDtypeStruct((nd, cpad), jnp.float32),
        scratch_shapes=[pltpu.VMEM((gb * _KP * _GP, cin), jnp.float32)],
    )(*args, kpt, Wflat)


def _decoder_body(xu_ref, sk_ref, wu_ref, bu_ref, wh_ref, bh_ref, ws_ref, bs_ref, o_ref):
    xc = jnp.concatenate([xu_ref[...], sk_ref[...][:, :_C1]], axis=1)
    x = _leaky(jnp.dot(xc, wu_ref[...], preferred_element_type=jnp.float32) + bu_ref[...])
    x = _leaky(jnp.dot(x, wh_ref[...], preferred_element_type=jnp.float32) + bh_ref[...])
    o_ref[...] = jnp.dot(x, ws_ref[...], preferred_element_type=jnp.float32) + bs_ref[...]


def _decoder(xu, skip, Wu, bu, Wh, bh, Ws, bs, bn):
    nd = xu.shape[0]
    grid = (nd // bn,)
    return pl.pallas_call(
        _decoder_body,
        grid=grid,
        in_specs=[
            pl.BlockSpec((bn, _C2), lambda i: (i, 0)),
            pl.BlockSpec((bn, 128), lambda i: (i, 0)),
            pl.BlockSpec((_C2 + _C1, _C1), lambda i: (0, 0)),
            pl.BlockSpec((1, _C1), lambda i: (0, 0)),
            pl.BlockSpec((_C1, _C1), lambda i: (0, 0)),
            pl.BlockSpec((1, _C1), lambda i: (0, 0)),
            pl.BlockSpec((_C1, _NCLS), lambda i: (0, 0)),
            pl.BlockSpec((1, _NCLS), lambda i: (0, 0)),
        ],
        out_specs=pl.BlockSpec((bn, _NCLS), lambda i: (i, 0)),
        out_shape=jax.ShapeDtypeStruct((nd, _NCLS), jnp.float32),
    )(xu, skip, Wu, bu, Wh, bh, Ws, bs)


def kernel(features, points0, points1, neighbors0, pools1, neighbors1,
           upsamples0, kp0, kp1, W1, W2, W3, Wu, bu, Wh, bh, Ws, bs):
    kpt0 = kp0.T.reshape(3 * _KP, 1)
    kpt1 = kp1.T.reshape(3 * _KP, 1)
    npad = 2560

    n0p = 10240
    p0x, p0y, p0z = points0[:, 0], points0[:, 1], points0[:, 2]
    p0xp = jnp.pad(p0x, (0, n0p - _N0))
    p0yp = jnp.pad(p0y, (0, n0p - _N0))
    p0zp = jnp.pad(p0z, (0, n0p - _N0))
    p1_p = jnp.pad(points1, ((0, npad - _N1), (0, 0)))
    p1x, p1y, p1z = p1_p[:, 0], p1_p[:, 1], p1_p[:, 2]

    def dst_exp(c):
        return jnp.repeat(c, _KN)

    idx0 = jnp.pad(neighbors0.reshape(-1).astype(jnp.int32),
                   (0, (n0p - _N0) * _KN))
    nx0, gx, gy, gz = _sc_gather_layer(features, p0x, p0y, p0z, idx0, chunk=128)
    W1f = W1.reshape(_KP * _CIN, _C1)
    x0 = _kpconv(nx0, gx, gy, gz, dst_exp(p0xp), dst_exp(p0yp), dst_exp(p0zp),
                 kpt0, W1f, _EXT0, bn=256, cpad=128)

    pools_p = jnp.pad(pools1, ((0, npad - _N1), (0, 0)))
    idx1 = pools_p.reshape(-1).astype(jnp.int32)
    nx1, gx, gy, gz = _sc_gather_layer(x0, p0x, p0y, p0z, idx1, chunk=128)
    W2p = jnp.pad(W2, ((0, 0), (0, 128 - _C1), (0, 0))).reshape(_KP * 128, _C2)
    x1 = _kpconv(nx1, gx, gy, gz, dst_exp(p1x), dst_exp(p1y), dst_exp(p1z),
                 kpt0, W2p, _EXT0, bn=256)

    neigh1_p = jnp.pad(neighbors1, ((0, npad - _N1), (0, 0)))
    idx2 = neigh1_p.reshape(-1).astype(jnp.int32)
    nx2, gx, gy, gz = _sc_gather_layer(x1, p1x, p1y, p1z, idx2, chunk=128)
    W3f = W3.reshape(_KP * _C2, _C2)
    x1 = _kpconv(nx2, gx, gy, gz, dst_exp(p1x), dst_exp(p1y), dst_exp(p1z),
                 kpt1, W3f, _EXT1, bn=256)

    idxu = jnp.pad(upsamples0[:, 0], (0, n0p - _N0)).astype(jnp.int32)
    xu = _sc_gather_rows(x1, idxu)
    logits = _decoder(xu, x0, Wu, bu.reshape(1, -1), Wh, bh.reshape(1, -1),
                      Ws, bs.reshape(1, -1), bn=1024)
    return logits[:_N0]


# speedup vs baseline: 1.0168x; 1.0168x over previous
import functools

import jax
import jax.numpy as jnp
from jax import lax
from jax.experimental import pallas as pl
from jax.experimental.pallas import tpu as pltpu
from jax.experimental.pallas import tpu_sc as plsc

_N0, _N1, _KN, _KP = 10000, 2500, 32, 15
_CIN, _C1, _C2, _NCLS = 128, 64, 128, 19
_EXT0, _EXT1 = 0.05, 0.1

_NW = 32


def _sc_gather_layer(feat, ptsx, ptsy, ptsz, idx_flat, chunk):
    B = idx_flat.shape[0]
    D = feat.shape[1]
    V = ptsx.shape[0]
    b_per_w = B // _NW
    nchunk = b_per_w // chunk
    mesh = plsc.VectorSubcoreMesh(core_axis_name="c", subcore_axis_name="s")
    f32 = jnp.float32

    @functools.partial(
        pl.kernel,
        mesh=mesh,
        compiler_params=pltpu.CompilerParams(needs_layout_passes=False),
        out_type=[
            jax.ShapeDtypeStruct((B, D), f32),
            jax.ShapeDtypeStruct((B,), f32),
            jax.ShapeDtypeStruct((B,), f32),
            jax.ShapeDtypeStruct((B,), f32),
        ],
        scratch_types=[
            pltpu.VMEM((b_per_w,), jnp.int32),
            pltpu.VMEM((chunk, D), f32),
            pltpu.VMEM((chunk, D), f32),
            pltpu.VMEM((b_per_w,), f32),
            pltpu.VMEM((b_per_w,), f32),
            pltpu.VMEM((b_per_w,), f32),
            pltpu.VMEM((V,), f32),
            pltpu.VMEM((V,), f32),
            pltpu.VMEM((V,), f32),
            pltpu.SemaphoreType.DMA,
            pltpu.SemaphoreType.DMA,
        ],
    )
    def k(feat_h, px_h, py_h, pz_h, idx_h, of_h, ox_h, oy_h, oz_h,
          idx_v, rows_a, rows_b, xv, yv, zv, ptx_v, pty_v, ptz_v, sema, semb):
        assert nchunk % 2 == 0 and nchunk >= 4
        wid = lax.axis_index("s") * 2 + lax.axis_index("c")
        base = wid * b_per_w
        pltpu.sync_copy(idx_h.at[pl.ds(base, b_per_w)], idx_v)
        pltpu.sync_copy(px_h, ptx_v)
        pltpu.sync_copy(py_h, pty_v)
        pltpu.sync_copy(pz_h, ptz_v)

        def fire(ci, buf, sem):
            pltpu.async_copy(feat_h.at[idx_v.at[pl.ds(ci * chunk, chunk)]], buf, sem)

        def wait_g(buf, sem):
            pltpu.make_async_copy(feat_h.at[pl.ds(0, chunk)], buf, sem).wait()

        def wb(ci, buf):
            pltpu.sync_copy(buf, of_h.at[pl.ds(base + ci * chunk, chunk)])

        fire(0, rows_a, sema)

        def pbody(j, _):
            ivec = idx_v[pl.ds(j * 16, 16)]
            xv[pl.ds(j * 16, 16)] = plsc.load_gather(ptx_v, [ivec])
            yv[pl.ds(j * 16, 16)] = plsc.load_gather(pty_v, [ivec])
            zv[pl.ds(j * 16, 16)] = plsc.load_gather(ptz_v, [ivec])
            return ()

        lax.fori_loop(0, b_per_w // 16, pbody, ())

        def body(i, _):
            c = 2 * i
            fire(c + 1, rows_b, semb)
            wait_g(rows_a, sema)
            wb(c, rows_a)

            @pl.when(c + 2 < nchunk)
            def _():
                fire(c + 2, rows_a, sema)

            wait_g(rows_b, semb)
            wb(c + 1, rows_b)
            return ()

        lax.fori_loop(0, nchunk // 2, body, ())

        pltpu.sync_copy(xv, ox_h.at[pl.ds(base, b_per_w)])
        pltpu.sync_copy(yv, oy_h.at[pl.ds(base, b_per_w)])
        pltpu.sync_copy(zv, oz_h.at[pl.ds(base, b_per_w)])

    return k(feat, ptsx, ptsy, ptsz, idx_flat)


def _sc_gather_rows(table, idx):
    B = idx.shape[0]
    D = table.shape[1]
    b_per_w = B // _NW
    chunk = 80
    nchunk = b_per_w // chunk
    mesh = plsc.VectorSubcoreMesh(core_axis_name="c", subcore_axis_name="s")

    @functools.partial(
        pl.kernel,
        mesh=mesh,
        compiler_params=pltpu.CompilerParams(needs_layout_passes=False),
        out_type=jax.ShapeDtypeStruct((B, D), jnp.float32),
        scratch_types=[
            pltpu.VMEM((chunk,), jnp.int32),
            pltpu.VMEM((chunk, D), jnp.float32),
            pltpu.SemaphoreType.DMA,
        ],
    )
    def k(tab_h, idx_h, out_h, idx_v, rows_v, sem):
        wid = lax.axis_index("s") * 2 + lax.axis_index("c")
        base = wid * b_per_w

        def body(i, _):
            off = base + i * chunk
            pltpu.sync_copy(idx_h.at[pl.ds(off, chunk)], idx_v)
            pltpu.async_copy(tab_h.at[idx_v], rows_v, sem).wait()
            pltpu.sync_copy(rows_v, out_h.at[pl.ds(off, chunk)])
            return ()

        lax.fori_loop(0, nchunk, body, ())

    return k(table, idx)


def _leaky(x):
    return jnp.where(x > 0, x, 0.1 * x)


_GP = 8
_GW = _GP * _KN


def _kpconv_body(nx_ref, px_ref, py_ref, pz_ref, dx_ref, dy_ref, dz_ref,
                 kpt_ref, w_ref, o_ref, wf_scr, *, ext, bn, cpad):
    cin = nx_ref.shape[-1]
    cout = w_ref.shape[-1]
    ngrp = bn // _GP
    kx = kpt_ref[0:_KP, :]
    ky = kpt_ref[_KP:2 * _KP, :]
    kz = kpt_ref[2 * _KP:3 * _KP, :]
    pxr = px_ref[...] - dx_ref[...]
    pyr = py_ref[...] - dy_ref[...]
    pzr = pz_ref[...] - dz_ref[...]
    rows = _KP * _GP
    r8 = lax.broadcasted_iota(jnp.int32, (rows, _GW), 0) % _GP
    c32 = lax.broadcasted_iota(jnp.int32, (rows, _GW), 1) // _KN
    maskf = (r8 == c32).astype(jnp.float32)
    for g in range(ngrp):
        ax = pxr[g:g + 1, :] - kx
        ay = pyr[g:g + 1, :] - ky
        az = pzr[g:g + 1, :] - kz
        d2 = ax * ax + ay * ay + az * az
        w15 = jnp.maximum(0.0, 1.0 - jnp.sqrt(d2) * (1.0 / ext))
        wrep = jnp.broadcast_to(w15[:, None, :], (_KP, _GP, _GW)).reshape(rows, _GW)
        m = wrep * maskf
        nxg = nx_ref[g * _GW:(g + 1) * _GW, :]
        wf_scr[g * rows:(g + 1) * rows, :] = jnp.dot(
            m, nxg, preferred_element_type=jnp.float32)
    acc = jnp.zeros((bn, cout), jnp.float32)
    for p in range(_KP):
        parts = [wf_scr[g * rows + p * _GP: g * rows + (p + 1) * _GP, :]
                 for g in range(ngrp)]
        wfp = jnp.concatenate(parts, axis=0)
        acc = acc + jnp.dot(wfp, w_ref[p * cin:(p + 1) * cin, :],
                            preferred_element_type=jnp.float32)
    res = _leaky(acc)
    if cpad > res.shape[1]:
        res = jnp.concatenate(
            [res, jnp.zeros((bn, cpad - res.shape[1]), jnp.float32)], axis=1)
    o_ref[...] = res


def _kpconv(nx_flat, px, py, pz, dxe, dye, dze, kpt, Wflat, ext, bn, cpad=None):
    nd = nx_flat.shape[0] // _KN
    cin = nx_flat.shape[-1]
    cout = Wflat.shape[-1]
    if cpad is None:
        cpad = cout
    grid = (nd // bn,)
    gb = bn // _GP
    pr = (nd // _GP, _GW)
    args = [nx_flat] + [a.reshape(pr) for a in (px, py, pz, dxe, dye, dze)]
    return pl.pallas_call(
        functools.partial(_kpconv_body, ext=ext, bn=bn, cpad=cpad),
        grid=grid,
        in_specs=[pl.BlockSpec((bn * _KN, cin), lambda i: (i, 0))]
        + [pl.BlockSpec((gb, _GW), lambda i: (i, 0))] * 6
        + [
            pl.BlockSpec((3 * _KP, 1), lambda i: (0, 0)),
            pl.BlockSpec((_KP * cin, cout), lambda i: (0, 0)),
        ],
        out_specs=pl.BlockSpec((bn, cpad), lambda i: (i, 0)),
        out_shape=jax.ShapeDtypeStruct((nd, cpad), jnp.float32),
        scratch_shapes=[pltpu.VMEM((gb * _KP * _GP, cin), jnp.float32)],
    )(*args, kpt, Wflat)


def _decoder_body(xu_ref, sk_ref, wu_ref, bu_ref, wh_ref, bh_ref, ws_ref, bs_ref, o_ref):
    xc = jnp.concatenate([xu_ref[...], sk_ref[...][:, :_C1]], axis=1)
    x = _leaky(jnp.dot(xc, wu_ref[...], preferred_element_type=jnp.float32) + bu_ref[...])
    x = _leaky(jnp.dot(x, wh_ref[...], preferred_element_type=jnp.float32) + bh_ref[...])
    o_ref[...] = jnp.dot(x, ws_ref[...], preferred_element_type=jnp.float32) + bs_ref[...]


def _decoder(xu, skip, Wu, bu, Wh, bh, Ws, bs, bn):
    nd = xu.shape[0]
    grid = (nd // bn,)
    return pl.pallas_call(
        _decoder_body,
        grid=grid,
        in_specs=[
            pl.BlockSpec((bn, _C2), lambda i: (i, 0)),
            pl.BlockSpec((bn, 128), lambda i: (i, 0)),
            pl.BlockSpec((_C2 + _C1, _C1), lambda i: (0, 0)),
            pl.BlockSpec((1, _C1), lambda i: (0, 0)),
            pl.BlockSpec((_C1, _C1), lambda i: (0, 0)),
            pl.BlockSpec((1, _C1), lambda i: (0, 0)),
            pl.BlockSpec((_C1, _NCLS), lambda i: (0, 0)),
            pl.BlockSpec((1, _NCLS), lambda i: (0, 0)),
        ],
        out_specs=pl.BlockSpec((bn, _NCLS), lambda i: (i, 0)),
        out_shape=jax.ShapeDtypeStruct((nd, _NCLS), jnp.float32),
    )(xu, skip, Wu, bu, Wh, bh, Ws, bs)


def kernel(features, points0, points1, neighbors0, pools1, neighbors1,
           upsamples0, kp0, kp1, W1, W2, W3, Wu, bu, Wh, bh, Ws, bs):
    kpt0 = kp0.T.reshape(3 * _KP, 1)
    kpt1 = kp1.T.reshape(3 * _KP, 1)
    npad = 2560

    n0p = 10240
    p0x, p0y, p0z = points0[:, 0], points0[:, 1], points0[:, 2]
    p0xp = jnp.pad(p0x, (0, n0p - _N0))
    p0yp = jnp.pad(p0y, (0, n0p - _N0))
    p0zp = jnp.pad(p0z, (0, n0p - _N0))
    p1_p = jnp.pad(points1, ((0, npad - _N1), (0, 0)))
    p1x, p1y, p1z = p1_p[:, 0], p1_p[:, 1], p1_p[:, 2]

    def dst_exp(c):
        return jnp.repeat(c, _KN)

    idx0 = jnp.pad(neighbors0.reshape(-1).astype(jnp.int32),
                   (0, (n0p - _N0) * _KN))
    nx0, gx, gy, gz = _sc_gather_layer(features, p0x, p0y, p0z, idx0, chunk=128)
    W1f = W1.reshape(_KP * _CIN, _C1)
    x0 = _kpconv(nx0, gx, gy, gz, dst_exp(p0xp), dst_exp(p0yp), dst_exp(p0zp),
                 kpt0, W1f, _EXT0, bn=256, cpad=128)

    pools_p = jnp.pad(pools1, ((0, npad - _N1), (0, 0)))
    idx1 = pools_p.reshape(-1).astype(jnp.int32)
    nx1, gx, gy, gz = _sc_gather_layer(x0, p0x, p0y, p0z, idx1, chunk=80)
    W2p = jnp.pad(W2, ((0, 0), (0, 128 - _C1), (0, 0))).reshape(_KP * 128, _C2)
    x1 = _kpconv(nx1, gx, gy, gz, dst_exp(p1x), dst_exp(p1y), dst_exp(p1z),
                 kpt0, W2p, _EXT0, bn=256)

    neigh1_p = jnp.pad(neighbors1, ((0, npad - _N1), (0, 0)))
    idx2 = neigh1_p.reshape(-1).astype(jnp.int32)
    nx2, gx, gy, gz = _sc_gather_layer(x1, p1x, p1y, p1z, idx2, chunk=80)
    W3f = W3.reshape(_KP * _C2, _C2)
    x1 = _kpconv(nx2, gx, gy, gz, dst_exp(p1x), dst_exp(p1y), dst_exp(p1z),
                 kpt1, W3f, _EXT1, bn=256)

    idxu = jnp.pad(upsamples0[:, 0], (0, n0p - _N0)).astype(jnp.int32)
    xu = _sc_gather_rows(x1, idxu)
    logits = _decoder(xu, x0, Wu, bu.reshape(1, -1), Wh, bh.reshape(1, -1),
                      Ws, bs.reshape(1, -1), bn=1024)
    return logits[:_N0]


# speedup vs baseline: 1.0247x; 1.0078x over previous
import functools

import jax
import jax.numpy as jnp
from jax import lax
from jax.experimental import pallas as pl
from jax.experimental.pallas import tpu as pltpu
from jax.experimental.pallas import tpu_sc as plsc

_N0, _N1, _KN, _KP = 10000, 2500, 32, 15
_CIN, _C1, _C2, _NCLS = 128, 64, 128, 19
_EXT0, _EXT1 = 0.05, 0.1

_NW = 32


def _sc_gather_layer(feat, ptsx, ptsy, ptsz, idx_flat, chunk):
    B = idx_flat.shape[0]
    D = feat.shape[1]
    V = ptsx.shape[0]
    b_per_w = B // _NW
    nchunk = b_per_w // chunk
    mesh = plsc.VectorSubcoreMesh(core_axis_name="c", subcore_axis_name="s")
    f32 = jnp.float32

    @functools.partial(
        pl.kernel,
        mesh=mesh,
        compiler_params=pltpu.CompilerParams(needs_layout_passes=False),
        out_type=[
            jax.ShapeDtypeStruct((B, D), f32),
            jax.ShapeDtypeStruct((B,), f32),
            jax.ShapeDtypeStruct((B,), f32),
            jax.ShapeDtypeStruct((B,), f32),
        ],
        scratch_types=[
            pltpu.VMEM((b_per_w,), jnp.int32),
            pltpu.VMEM((chunk, D), f32),
            pltpu.VMEM((chunk, D), f32),
            pltpu.VMEM((b_per_w,), f32),
            pltpu.VMEM((b_per_w,), f32),
            pltpu.VMEM((b_per_w,), f32),
            pltpu.VMEM((V,), f32),
            pltpu.VMEM((V,), f32),
            pltpu.VMEM((V,), f32),
            pltpu.SemaphoreType.DMA,
            pltpu.SemaphoreType.DMA,
        ],
    )
    def k(feat_h, px_h, py_h, pz_h, idx_h, of_h, ox_h, oy_h, oz_h,
          idx_v, rows_a, rows_b, xv, yv, zv, ptx_v, pty_v, ptz_v, sema, semb):
        assert nchunk % 2 == 0 and nchunk >= 4
        wid = lax.axis_index("s") * 2 + lax.axis_index("c")
        base = wid * b_per_w
        pltpu.sync_copy(idx_h.at[pl.ds(base, b_per_w)], idx_v)
        pltpu.sync_copy(px_h, ptx_v)
        pltpu.sync_copy(py_h, pty_v)
        pltpu.sync_copy(pz_h, ptz_v)

        def fire(ci, buf, sem):
            pltpu.async_copy(feat_h.at[idx_v.at[pl.ds(ci * chunk, chunk)]], buf, sem)

        def wait_g(buf, sem):
            pltpu.make_async_copy(feat_h.at[pl.ds(0, chunk)], buf, sem).wait()

        def wb(ci, buf):
            pltpu.sync_copy(buf, of_h.at[pl.ds(base + ci * chunk, chunk)])

        def pos_chunk(ci):
            for j in range(chunk // 16):
                o = ci * chunk + j * 16
                ivec = idx_v[pl.ds(o, 16)]
                xv[pl.ds(o, 16)] = plsc.load_gather(ptx_v, [ivec])
                yv[pl.ds(o, 16)] = plsc.load_gather(pty_v, [ivec])
                zv[pl.ds(o, 16)] = plsc.load_gather(ptz_v, [ivec])

        fire(0, rows_a, sema)

        def body(i, _):
            c = 2 * i
            fire(c + 1, rows_b, semb)
            pos_chunk(c)
            wait_g(rows_a, sema)
            wb(c, rows_a)

            @pl.when(c + 2 < nchunk)
            def _():
                fire(c + 2, rows_a, sema)

            pos_chunk(c + 1)
            wait_g(rows_b, semb)
            wb(c + 1, rows_b)
            return ()

        lax.fori_loop(0, nchunk // 2, body, ())

        pltpu.sync_copy(xv, ox_h.at[pl.ds(base, b_per_w)])
        pltpu.sync_copy(yv, oy_h.at[pl.ds(base, b_per_w)])
        pltpu.sync_copy(zv, oz_h.at[pl.ds(base, b_per_w)])

    return k(feat, ptsx, ptsy, ptsz, idx_flat)


def _sc_gather_rows(table, idx):
    B = idx.shape[0]
    D = table.shape[1]
    b_per_w = B // _NW
    chunk = 80
    nchunk = b_per_w // chunk
    mesh = plsc.VectorSubcoreMesh(core_axis_name="c", subcore_axis_name="s")

    @functools.partial(
        pl.kernel,
        mesh=mesh,
        compiler_params=pltpu.CompilerParams(needs_layout_passes=False),
        out_type=jax.ShapeDtypeStruct((B, D), jnp.float32),
        scratch_types=[
            pltpu.VMEM((chunk,), jnp.int32),
            pltpu.VMEM((chunk, D), jnp.float32),
            pltpu.SemaphoreType.DMA,
        ],
    )
    def k(tab_h, idx_h, out_h, idx_v, rows_v, sem):
        wid = lax.axis_index("s") * 2 + lax.axis_index("c")
        base = wid * b_per_w

        def body(i, _):
            off = base + i * chunk
            pltpu.sync_copy(idx_h.at[pl.ds(off, chunk)], idx_v)
            pltpu.async_copy(tab_h.at[idx_v], rows_v, sem).wait()
            pltpu.sync_copy(rows_v, out_h.at[pl.ds(off, chunk)])
            return ()

        lax.fori_loop(0, nchunk, body, ())

    return k(table, idx)


def _leaky(x):
    return jnp.where(x > 0, x, 0.1 * x)


_GP = 8
_GW = _GP * _KN


def _kpconv_body(nx_ref, px_ref, py_ref, pz_ref, dx_ref, dy_ref, dz_ref,
                 kpt_ref, w_ref, o_ref, wf_scr, *, ext, bn, cpad):
    cin = nx_ref.shape[-1]
    cout = w_ref.shape[-1]
    ngrp = bn // _GP
    kx = kpt_ref[0:_KP, :]
    ky = kpt_ref[_KP:2 * _KP, :]
    kz = kpt_ref[2 * _KP:3 * _KP, :]
    pxr = px_ref[...] - dx_ref[...]
    pyr = py_ref[...] - dy_ref[...]
    pzr = pz_ref[...] - dz_ref[...]
    rows = _KP * _GP
    r8 = lax.broadcasted_iota(jnp.int32, (rows, _GW), 0) % _GP
    c32 = lax.broadcasted_iota(jnp.int32, (rows, _GW), 1) // _KN
    maskf = (r8 == c32).astype(jnp.float32)
    for g in range(ngrp):
        ax = pxr[g:g + 1, :] - kx
        ay = pyr[g:g + 1, :] - ky
        az = pzr[g:g + 1, :] - kz
        d2 = ax * ax + ay * ay + az * az
        w15 = jnp.maximum(0.0, 1.0 - jnp.sqrt(d2) * (1.0 / ext))
        wrep = jnp.broadcast_to(w15[:, None, :], (_KP, _GP, _GW)).reshape(rows, _GW)
        m = wrep * maskf
        nxg = nx_ref[g * _GW:(g + 1) * _GW, :]
        wf_scr[g * rows:(g + 1) * rows, :] = jnp.dot(
            m, nxg, preferred_element_type=jnp.float32)
    acc = jnp.zeros((bn, cout), jnp.float32)
    for p in range(_KP):
        parts = [wf_scr[g * rows + p * _GP: g * rows + (p + 1) * _GP, :]
                 for g in range(ngrp)]
        wfp = jnp.concatenate(parts, axis=0)
        acc = acc + jnp.dot(wfp, w_ref[p * cin:(p + 1) * cin, :],
                            preferred_element_type=jnp.float32)
    res = _leaky(acc)
    if cpad > res.shape[1]:
        res = jnp.concatenate(
            [res, jnp.zeros((bn, cpad - res.shape[1]), jnp.float32)], axis=1)
    o_ref[...] = res


def _kpconv(nx_flat, px, py, pz, dxe, dye, dze, kpt, Wflat, ext, bn, cpad=None):
    nd = nx_flat.shape[0] // _KN
    cin = nx_flat.shape[-1]
    cout = Wflat.shape[-1]
    if cpad is None:
        cpad = cout
    grid = (nd // bn,)
    gb = bn // _GP
    pr = (nd // _GP, _GW)
    args = [nx_flat] + [a.reshape(pr) for a in (px, py, pz, dxe, dye, dze)]
    return pl.pallas_call(
        functools.partial(_kpconv_body, ext=ext, bn=bn, cpad=cpad),
        grid=grid,
        in_specs=[pl.BlockSpec((bn * _KN, cin), lambda i: (i, 0))]
        + [pl.BlockSpec((gb, _GW), lambda i: (i, 0))] * 6
        + [
            pl.BlockSpec((3 * _KP, 1), lambda i: (0, 0)),
            pl.BlockSpec((_KP * cin, cout), lambda i: (0, 0)),
        ],
        out_specs=pl.BlockSpec((bn, cpad), lambda i: (i, 0)),
        out_shape=jax.ShapeDtypeStruct((nd, cpad), jnp.float32),
        scratch_shapes=[pltpu.VMEM((gb * _KP * _GP, cin), jnp.float32)],
    )(*args, kpt, Wflat)


def _decoder_body(xu_ref, sk_ref, wu_ref, bu_ref, wh_ref, bh_ref, ws_ref, bs_ref, o_ref):
    xc = jnp.concatenate([xu_ref[...], sk_ref[...][:, :_C1]], axis=1)
    x = _leaky(jnp.dot(xc, wu_ref[...], preferred_element_type=jnp.float32) + bu_ref[...])
    x = _leaky(jnp.dot(x, wh_ref[...], preferred_element_type=jnp.float32) + bh_ref[...])
    o_ref[...] = jnp.dot(x, ws_ref[...], preferred_element_type=jnp.float32) + bs_ref[...]


def _decoder(xu, skip, Wu, bu, Wh, bh, Ws, bs, bn):
    nd = xu.shape[0]
    grid = (nd // bn,)
    return pl.pallas_call(
        _decoder_body,
        grid=grid,
        in_specs=[
            pl.BlockSpec((bn, _C2), lambda i: (i, 0)),
            pl.BlockSpec((bn, 128), lambda i: (i, 0)),
            pl.BlockSpec((_C2 + _C1, _C1), lambda i: (0, 0)),
            pl.BlockSpec((1, _C1), lambda i: (0, 0)),
            pl.BlockSpec((_C1, _C1), lambda i: (0, 0)),
            pl.BlockSpec((1, _C1), lambda i: (0, 0)),
            pl.BlockSpec((_C1, _NCLS), lambda i: (0, 0)),
            pl.BlockSpec((1, _NCLS), lambda i: (0, 0)),
        ],
        out_specs=pl.BlockSpec((bn, _NCLS), lambda i: (i, 0)),
        out_shape=jax.ShapeDtypeStruct((nd, _NCLS), jnp.float32),
    )(xu, skip, Wu, bu, Wh, bh, Ws, bs)


def kernel(features, points0, points1, neighbors0, pools1, neighbors1,
           upsamples0, kp0, kp1, W1, W2, W3, Wu, bu, Wh, bh, Ws, bs):
    kpt0 = kp0.T.reshape(3 * _KP, 1)
    kpt1 = kp1.T.reshape(3 * _KP, 1)
    npad = 2560

    n0p = 10240
    p0x, p0y, p0z = points0[:, 0], points0[:, 1], points0[:, 2]
    p0xp = jnp.pad(p0x, (0, n0p - _N0))
    p0yp = jnp.pad(p0y, (0, n0p - _N0))
    p0zp = jnp.pad(p0z, (0, n0p - _N0))
    p1_p = jnp.pad(points1, ((0, npad - _N1), (0, 0)))
    p1x, p1y, p1z = p1_p[:, 0], p1_p[:, 1], p1_p[:, 2]

    def dst_exp(c):
        return jnp.repeat(c, _KN)

    idx0 = jnp.pad(neighbors0.reshape(-1).astype(jnp.int32),
                   (0, (n0p - _N0) * _KN))
    nx0, gx, gy, gz = _sc_gather_layer(features, p0x, p0y, p0z, idx0, chunk=128)
    W1f = W1.reshape(_KP * _CIN, _C1)
    x0 = _kpconv(nx0, gx, gy, gz, dst_exp(p0xp), dst_exp(p0yp), dst_exp(p0zp),
                 kpt0, W1f, _EXT0, bn=256, cpad=128)

    pools_p = jnp.pad(pools1, ((0, npad - _N1), (0, 0)))
    idx1 = pools_p.reshape(-1).astype(jnp.int32)
    nx1, gx, gy, gz = _sc_gather_layer(x0, p0x, p0y, p0z, idx1, chunk=80)
    W2p = jnp.pad(W2, ((0, 0), (0, 128 - _C1), (0, 0))).reshape(_KP * 128, _C2)
    x1 = _kpconv(nx1, gx, gy, gz, dst_exp(p1x), dst_exp(p1y), dst_exp(p1z),
                 kpt0, W2p, _EXT0, bn=256)

    neigh1_p = jnp.pad(neighbors1, ((0, npad - _N1), (0, 0)))
    idx2 = neigh1_p.reshape(-1).astype(jnp.int32)
    nx2, gx, gy, gz = _sc_gather_layer(x1, p1x, p1y, p1z, idx2, chunk=80)
    W3f = W3.reshape(_KP * _C2, _C2)
    x1 = _kpconv(nx2, gx, gy, gz, dst_exp(p1x), dst_exp(p1y), dst_exp(p1z),
                 kpt1, W3f, _EXT1, bn=256)

    idxu = jnp.pad(upsamples0[:, 0], (0, n0p - _N0)).astype(jnp.int32)
    xu = _sc_gather_rows(x1, idxu)
    logits = _decoder(xu, x0, Wu, bu.reshape(1, -1), Wh, bh.reshape(1, -1),
                      Ws, bs.reshape(1, -1), bn=1024)
    return logits[:_N0]
